# baseline (speedup 1.0000x reference)
import functools

import jax
import jax.numpy as jnp
from jax import lax
from jax.experimental import pallas as pl
from jax.experimental.pallas import tpu as pltpu
from jax.experimental.pallas import tpu_sc as plsc

VOCAB = 1000000
EMBED_DIM = 64
PADDED_DIM = 128
BATCH = 4096
HIST = 200

NUM_CORES = 2
NUM_SUBCORES = 16
NW = NUM_CORES * NUM_SUBCORES

TOT = BATCH * HIST
PER_W = TOT // NW
CHUNK = 128
NCH = PER_W // CHUNK
NBUF = 4

_MESH = plsc.VectorSubcoreMesh(core_axis_name="c", subcore_axis_name="s")


def _worker_id():
    return lax.axis_index("s") * NUM_CORES + lax.axis_index("c")


@functools.partial(
    pl.kernel,
    out_type=jax.ShapeDtypeStruct((TOT, PADDED_DIM), jnp.float32),
    mesh=_MESH,
    compiler_params=pltpu.CompilerParams(use_tc_tiling_on_sc=False),
    scratch_types=[
        pltpu.VMEM((NCH, CHUNK), jnp.int32),
        *[pltpu.VMEM((CHUNK, EMBED_DIM), jnp.float32) for _ in range(NBUF)],
        *[pltpu.SemaphoreType.DMA for _ in range(NBUF)],
    ],
)
def _sc_gather(idx_hbm, table_hbm, out_hbm, idx_v, *bufs_and_sems):
    bufs = bufs_and_sems[:NBUF]
    sems = bufs_and_sems[NBUF:]

    wid = _worker_id()
    chunk0 = wid * NCH

    pltpu.sync_copy(idx_hbm.at[pl.ds(chunk0, NCH)], idx_v)

    for b in range(NBUF):
        pltpu.async_copy(table_hbm.at[idx_v.at[b]], bufs[b], sems[b])

    def body(g, _):
        for b in range(NBUF):
            j = g * NBUF + b
            pltpu.make_async_copy(
                table_hbm.at[idx_v.at[j]], bufs[b], sems[b]
            ).wait()
            pltpu.sync_copy(
                bufs[b],
                out_hbm.at[pl.ds((chunk0 + j) * CHUNK, CHUNK), pl.ds(0, EMBED_DIM)],
            )

            @pl.when(j + NBUF < NCH)
            def _():
                pltpu.async_copy(
                    table_hbm.at[idx_v.at[j + NBUF]], bufs[b], sems[b]
                )

        return 0

    lax.fori_loop(0, NCH // NBUF, body, 0)


@jax.jit
def kernel(input_indices, table):
    table2 = jnp.pad(table, ((0, 0), (0, PADDED_DIM - EMBED_DIM)))
    table2 = table2.reshape(2 * VOCAB, EMBED_DIM)
    idx = (input_indices * 2).reshape(TOT // CHUNK, CHUNK)
    out = _sc_gather(idx, table2)
    return out[:, :EMBED_DIM].reshape(BATCH, HIST, EMBED_DIM)
